# baseline (speedup 1.0000x reference)
import functools
import jax
import jax.numpy as jnp
from jax import lax
from jax.experimental import pallas as pl
from jax.experimental.pallas import tpu as pltpu
from jax.experimental.pallas import tpu_sc as plsc

H = 512
W = 512
NB = H * W
NPTS = 131072
BATCH = 4
CH = 8
NC = 2
NS = 16
CHUNK = 1024
PTS_PER_TILE = NPTS // NS
CHUNKS_PER_TILE = PTS_PER_TILE // CHUNK
GROUPS = CHUNK // 16
BINS_PER_TILE = NB // NS
NSUB = 4096
MAGIC = 12582912.0


def _sc_body(x_hbm, c_hbm, idx_hbm, out_hbm,
             acc0, acc1, acc2, acc3, acc4,
             x_v, c_v, i_v, idx_v, v0, v1, v2, v3, v4, cnt_v, ch_v, zc_v, sem):
    core = lax.axis_index("c")
    sid = lax.axis_index("s")
    accs = (acc0, acc1, acc2, acc3, acc4)
    vals = (v0, v1, v2, v3, v4)

    lanes = lax.iota(jnp.int32, 16)
    zeros16 = lanes * 0
    ones16 = zeros16 + 1
    twos16 = zeros16 + 2

    nb_base = sid * BINS_PER_TILE

    def _zc(i, _):
        zc_v[pl.ds(i * 16, 16)] = jnp.zeros((16,), jnp.float32)
        return 0
    lax.fori_loop(0, NSUB // 16, _zc, 0)
    for p in range(5):
        for sub in range(BINS_PER_TILE // NSUB):
            pltpu.sync_copy(
                zc_v, accs[p].at[pl.ds(nb_base + sub * NSUB, NSUB)])
    plsc.subcore_barrier()

    def _batch(b, _):
        def _chunk(chunk, _):
            base = sid * PTS_PER_TILE + chunk * CHUNK
            pltpu.sync_copy(x_hbm.at[b, pl.ds(base, CHUNK)], x_v)
            pltpu.sync_copy(c_hbm.at[b, pl.ds(base, CHUNK)], c_v)
            pltpu.sync_copy(idx_hbm.at[b, pl.ds(base, CHUNK)], i_v)

            def _grp(g, _):
                prow = g * 16 + lanes
                iu = plsc.load_gather(i_v, [prow, zeros16])
                iv = plsc.load_gather(i_v, [prow, ones16])
                x2 = plsc.load_gather(x_v, [prow, twos16])
                cond = ((iu > 0) & (iu < W) & (iv > 0) & (iv < H)
                        & (x2 > 0.0))
                ind = iu + iv * H
                gp = base + prow
                ind = jnp.where(cond, ind, gp)
                idx_v[pl.ds(g * 16, 16)] = ind
                v4[pl.ds(g * 16, 16)] = jnp.where(cond, 1.0, 0.0)
                for ch in range(4):
                    cval = plsc.load_gather(c_v, [prow, zeros16 + (core * 4 + ch)])
                    vals[ch][pl.ds(g * 16, 16)] = jnp.where(cond, cval, 0.0)
                return 0
            lax.fori_loop(0, GROUPS, _grp, 0)

            copies = []
            for p in range(5):
                copies.append(pltpu.async_copy(
                    vals[p], accs[p].at[idx_v], sem, add=True))
            for cp in copies:
                cp.wait()
            return 0
        lax.fori_loop(0, CHUNKS_PER_TILE, _chunk, 0)
        plsc.subcore_barrier()

        def _sub(sub, _):
            sbase = nb_base + sub * NSUB
            pltpu.sync_copy(accs[4].at[pl.ds(sbase, NSUB)], cnt_v)

            def _rcp(i, _):
                c16 = cnt_v[pl.ds(i * 16, 16)]
                cnt_v[pl.ds(i * 16, 16)] = 1.0 / jnp.maximum(c16, 1.0)
                return 0
            lax.fori_loop(0, NSUB // 16, _rcp, 0)

            row_base = sbase // W
            for ch in range(4):
                pltpu.sync_copy(accs[ch].at[pl.ds(sbase, NSUB)], ch_v)

                def _mul(i, _):
                    ch_v[pl.ds(i * 16, 16)] = (
                        ch_v[pl.ds(i * 16, 16)] * cnt_v[pl.ds(i * 16, 16)])
                    return 0
                lax.fori_loop(0, NSUB // 16, _mul, 0)
                for r in range(NSUB // W):
                    pltpu.sync_copy(
                        ch_v.at[pl.ds(r * W, W)],
                        out_hbm.at[b, core * 4 + ch, row_base + r])
            for p in range(5):
                pltpu.sync_copy(zc_v, accs[p].at[pl.ds(sbase, NSUB)])
            return 0
        lax.fori_loop(0, BINS_PER_TILE // NSUB, _sub, 0)
        plsc.subcore_barrier()
        return 0
    lax.fori_loop(0, BATCH, _batch, 0)


@jax.jit
def kernel(x, c, K):
    cam = x / x[..., -1:]
    pix = jnp.matmul(cam, K.T)[..., :2]
    idx = jnp.round(jax.lax.stop_gradient(pix)).astype(jnp.int32)
    mesh = plsc.VectorSubcoreMesh(core_axis_name="c", subcore_axis_name="s")
    out = pl.kernel(
        _sc_body,
        out_type=jax.ShapeDtypeStruct((BATCH, CH, H, W), jnp.float32),
        mesh=mesh,
        compiler_params=pltpu.CompilerParams(needs_layout_passes=False, use_tc_tiling_on_sc=False),
        scratch_types=[
            pltpu.VMEM_SHARED((NB,), jnp.float32),
            pltpu.VMEM_SHARED((NB,), jnp.float32),
            pltpu.VMEM_SHARED((NB,), jnp.float32),
            pltpu.VMEM_SHARED((NB,), jnp.float32),
            pltpu.VMEM_SHARED((NB,), jnp.float32),
            pltpu.VMEM((CHUNK, 3), jnp.float32),
            pltpu.VMEM((CHUNK, 8), jnp.float32),
            pltpu.VMEM((CHUNK, 2), jnp.int32),
            pltpu.VMEM((CHUNK,), jnp.int32),
            pltpu.VMEM((CHUNK,), jnp.float32),
            pltpu.VMEM((CHUNK,), jnp.float32),
            pltpu.VMEM((CHUNK,), jnp.float32),
            pltpu.VMEM((CHUNK,), jnp.float32),
            pltpu.VMEM((CHUNK,), jnp.float32),
            pltpu.VMEM((NSUB,), jnp.float32),
            pltpu.VMEM((NSUB,), jnp.float32),
            pltpu.VMEM((NSUB,), jnp.float32),
            pltpu.SemaphoreType.DMA,
        ],
    )(x, c, idx)
    return out


# speedup vs baseline: 1.1275x; 1.1275x over previous
import functools
import jax
import jax.numpy as jnp
from jax import lax
from jax.experimental import pallas as pl
from jax.experimental.pallas import tpu as pltpu
from jax.experimental.pallas import tpu_sc as plsc

H = 512
W = 512
NB = H * W
NPTS = 131072
BATCH = 4
CH = 8
NC = 2
NS = 16
CHUNK = 1024
PTS_PER_TILE = NPTS // NS
CHUNKS_PER_TILE = PTS_PER_TILE // CHUNK
GROUPS = CHUNK // 16
BINS_PER_TILE = NB // NS
NSUB = 4096


def _sc_body(x_hbm, c_hbm, idx_hbm, out_hbm,
             acc0, acc1, acc2, acc3, acc4,
             x_v, c_v, i_v, idx_v, v0, v1, v2, v3, v4, cnt_v, ch_v, zc_v, sem):
    core = lax.axis_index("c")
    sid = lax.axis_index("s")
    accs = (acc0, acc1, acc2, acc3, acc4)
    vals = (v0, v1, v2, v3, v4)

    lanes = lax.iota(jnp.int32, 16)
    nb_base = sid * BINS_PER_TILE

    def _zc(i, _):
        zc_v[pl.ds(i * 16, 16)] = jnp.zeros((16,), jnp.float32)
        return 0
    lax.fori_loop(0, NSUB // 16, _zc, 0)
    for p in range(5):
        for sub in range(BINS_PER_TILE // NSUB):
            pltpu.sync_copy(
                zc_v, accs[p].at[pl.ds(nb_base + sub * NSUB, NSUB)])
    plsc.subcore_barrier()

    def _batch(b, _):
        def _chunk(chunk, _):
            base = sid * PTS_PER_TILE + chunk * CHUNK
            pbase = b * NPTS + base
            pltpu.sync_copy(x_hbm.at[pl.ds(pbase * 3, CHUNK * 3)], x_v)
            pltpu.sync_copy(c_hbm.at[pl.ds(pbase * 8, CHUNK * 8)], c_v)
            pltpu.sync_copy(idx_hbm.at[pl.ds(pbase * 2, CHUNK * 2)], i_v)

            def _grp(g, _):
                prow = g * 16 + lanes
                iu = plsc.load_gather(i_v, [prow * 2])
                iv = plsc.load_gather(i_v, [prow * 2 + 1])
                x2 = plsc.load_gather(x_v, [prow * 3 + 2])
                cond = ((iu > 0) & (iu < W) & (iv > 0) & (iv < H)
                        & (x2 > 0.0))
                ind = iu + iv * H
                gp = base + prow
                ind = jnp.where(cond, ind, gp)
                idx_v[pl.ds(g * 16, 16)] = ind
                v4[pl.ds(g * 16, 16)] = jnp.where(cond, 1.0, 0.0)
                for ch in range(4):
                    cval = plsc.load_gather(
                        c_v, [prow * 8 + (core * 4 + ch)])
                    vals[ch][pl.ds(g * 16, 16)] = jnp.where(cond, cval, 0.0)
                return 0
            lax.fori_loop(0, GROUPS, _grp, 0)

            copies = []
            for p in range(5):
                copies.append(pltpu.async_copy(
                    vals[p], accs[p].at[idx_v], sem, add=True))
            for cp in copies:
                cp.wait()
            return 0
        lax.fori_loop(0, CHUNKS_PER_TILE, _chunk, 0)
        plsc.subcore_barrier()

        def _sub(sub, _):
            sbase = nb_base + sub * NSUB
            pltpu.sync_copy(accs[4].at[pl.ds(sbase, NSUB)], cnt_v)

            def _rcp(i, _):
                c16 = cnt_v[pl.ds(i * 16, 16)]
                cnt_v[pl.ds(i * 16, 16)] = 1.0 / jnp.maximum(c16, 1.0)
                return 0
            lax.fori_loop(0, NSUB // 16, _rcp, 0)

            for ch in range(4):
                pltpu.sync_copy(accs[ch].at[pl.ds(sbase, NSUB)], ch_v)

                def _mul(i, _):
                    ch_v[pl.ds(i * 16, 16)] = (
                        ch_v[pl.ds(i * 16, 16)] * cnt_v[pl.ds(i * 16, 16)])
                    return 0
                lax.fori_loop(0, NSUB // 16, _mul, 0)
                pltpu.sync_copy(
                    ch_v,
                    out_hbm.at[pl.ds((b * CH + core * 4 + ch) * NB + sbase,
                                     NSUB)])
            for p in range(5):
                pltpu.sync_copy(zc_v, accs[p].at[pl.ds(sbase, NSUB)])
            return 0
        lax.fori_loop(0, BINS_PER_TILE // NSUB, _sub, 0)
        plsc.subcore_barrier()
        return 0
    lax.fori_loop(0, BATCH, _batch, 0)


ROWS_PER_BLK = 128


def _retile_body(in_ref, out_ref):
    for i in range(ROWS_PER_BLK):
        out_ref[0, 0, i, :] = in_ref[pl.ds(i * W, W)]


@jax.jit
def kernel(x, c, K):
    cam = x / x[..., -1:]
    pix = jnp.matmul(cam, K.T)[..., :2]
    idx = jnp.round(jax.lax.stop_gradient(pix)).astype(jnp.int32)

    mesh = plsc.VectorSubcoreMesh(core_axis_name="c", subcore_axis_name="s")
    out1 = pl.kernel(
        _sc_body,
        out_type=jax.ShapeDtypeStruct((BATCH * CH * NB,), jnp.float32),
        mesh=mesh,
        compiler_params=pltpu.CompilerParams(
            needs_layout_passes=False, use_tc_tiling_on_sc=False),
        scratch_types=[
            pltpu.VMEM_SHARED((NB,), jnp.float32),
            pltpu.VMEM_SHARED((NB,), jnp.float32),
            pltpu.VMEM_SHARED((NB,), jnp.float32),
            pltpu.VMEM_SHARED((NB,), jnp.float32),
            pltpu.VMEM_SHARED((NB,), jnp.float32),
            pltpu.VMEM((CHUNK * 3,), jnp.float32),
            pltpu.VMEM((CHUNK * 8,), jnp.float32),
            pltpu.VMEM((CHUNK * 2,), jnp.int32),
            pltpu.VMEM((CHUNK,), jnp.int32),
            pltpu.VMEM((CHUNK,), jnp.float32),
            pltpu.VMEM((CHUNK,), jnp.float32),
            pltpu.VMEM((CHUNK,), jnp.float32),
            pltpu.VMEM((CHUNK,), jnp.float32),
            pltpu.VMEM((CHUNK,), jnp.float32),
            pltpu.VMEM((NSUB,), jnp.float32),
            pltpu.VMEM((NSUB,), jnp.float32),
            pltpu.VMEM((NSUB,), jnp.float32),
            pltpu.SemaphoreType.DMA,
        ],
    )(x.reshape(-1), c.reshape(-1), idx.reshape(-1))

    out = pl.pallas_call(
        _retile_body,
        grid=(BATCH, CH, H // ROWS_PER_BLK),
        in_specs=[pl.BlockSpec(
            (ROWS_PER_BLK * W,),
            lambda b, ch, s: ((b * CH + ch) * (H // ROWS_PER_BLK) + s,))],
        out_specs=pl.BlockSpec(
            (1, 1, ROWS_PER_BLK, W), lambda b, ch, s: (b, ch, s, 0)),
        out_shape=jax.ShapeDtypeStruct((BATCH, CH, H, W), jnp.float32),
    )(out1)
    return out


# speedup vs baseline: 2.9735x; 2.6373x over previous
import functools
import jax
import jax.numpy as jnp
from jax import lax
from jax.experimental import pallas as pl
from jax.experimental.pallas import tpu as pltpu
from jax.experimental.pallas import tpu_sc as plsc

H = 512
W = 512
NB = H * W
NPTS = 131072
BATCH = 4
CH = 8
NC = 2
NS = 16
CHUNK = 1024
PTS_PER_TILE = NPTS // NS
CHUNKS_PER_TILE = PTS_PER_TILE // CHUNK
GROUPS = CHUNK // 16
BINS_PER_TILE = NB // NS
NSUB = 4096


def _sc_body(c_hbm, idx_hbm, out_hbm,
             acc0, acc1, acc2, acc3, acc4,
             c_v, i_v, idx_v, v0, v1, v2, v3, v4, cnt_v, ch_v, zc_v, sem):
    core = lax.axis_index("c")
    sid = lax.axis_index("s")
    accs = (acc0, acc1, acc2, acc3, acc4)
    vals = (v0, v1, v2, v3, v4)

    lanes = lax.iota(jnp.int32, 16)
    nb_base = sid * BINS_PER_TILE

    def _zc(i, _):
        zc_v[pl.ds(i * 16, 16)] = jnp.zeros((16,), jnp.float32)
        return 0
    lax.fori_loop(0, NSUB // 16, _zc, 0)
    for p in range(5):
        for sub in range(BINS_PER_TILE // NSUB):
            pltpu.sync_copy(
                zc_v, accs[p].at[pl.ds(nb_base + sub * NSUB, NSUB)])
    plsc.subcore_barrier()

    def _batch(b, _):
        def _chunk(chunk, _):
            base = sid * PTS_PER_TILE + chunk * CHUNK
            pbase = b * NPTS + base
            pltpu.sync_copy(c_hbm.at[pl.ds(pbase * 8, CHUNK * 8)], c_v)
            pltpu.sync_copy(idx_hbm.at[pl.ds(pbase * 2, CHUNK * 2)], i_v)

            def _grp(g, _):
                prow = g * 16 + lanes
                iu = plsc.load_gather(i_v, [prow * 2])
                iv = plsc.load_gather(i_v, [prow * 2 + 1])
                cond = (iu > 0) & (iu < W) & (iv > 0) & (iv < H)
                ind = iu + iv * H
                gp = base + prow
                ind = jnp.where(cond, ind, gp)
                idx_v[pl.ds(g * 16, 16)] = ind
                v4[pl.ds(g * 16, 16)] = jnp.where(cond, 1.0, 0.0)
                for ch in range(4):
                    cval = plsc.load_gather(
                        c_v, [prow * 8 + (core * 4 + ch)])
                    vals[ch][pl.ds(g * 16, 16)] = jnp.where(cond, cval, 0.0)
                return 0
            lax.fori_loop(0, GROUPS, _grp, 0)

            copies = []
            for p in range(5):
                copies.append(pltpu.async_copy(
                    vals[p], accs[p].at[idx_v], sem, add=True))
            for cp in copies:
                cp.wait()
            return 0
        lax.fori_loop(0, CHUNKS_PER_TILE, _chunk, 0)
        plsc.subcore_barrier()

        def _sub(sub, _):
            sbase = nb_base + sub * NSUB
            pltpu.sync_copy(accs[4].at[pl.ds(sbase, NSUB)], cnt_v)

            def _rcp(i, _):
                c16 = cnt_v[pl.ds(i * 16, 16)]
                cnt_v[pl.ds(i * 16, 16)] = 1.0 / jnp.maximum(c16, 1.0)
                return 0
            lax.fori_loop(0, NSUB // 16, _rcp, 0)

            for ch in range(4):
                pltpu.sync_copy(accs[ch].at[pl.ds(sbase, NSUB)], ch_v)

                def _mul(i, _):
                    ch_v[pl.ds(i * 16, 16)] = (
                        ch_v[pl.ds(i * 16, 16)] * cnt_v[pl.ds(i * 16, 16)])
                    return 0
                lax.fori_loop(0, NSUB // 16, _mul, 0)
                pltpu.sync_copy(
                    ch_v,
                    out_hbm.at[pl.ds((b * CH + core * 4 + ch) * NB + sbase,
                                     NSUB)])
            for p in range(5):
                pltpu.sync_copy(zc_v, accs[p].at[pl.ds(sbase, NSUB)])
            return 0
        lax.fori_loop(0, BINS_PER_TILE // NSUB, _sub, 0)
        plsc.subcore_barrier()
        return 0
    lax.fori_loop(0, BATCH, _batch, 0)


ROWS_PER_BLK = 128


def _retile_body(in_ref, out_ref):
    for i in range(ROWS_PER_BLK):
        out_ref[0, 0, i, :] = in_ref[pl.ds(i * W, W)]


@jax.jit
def kernel(x, c, K):
    cam = x / x[..., -1:]
    pix = jnp.matmul(cam, K.T)[..., :2]
    idx = jnp.round(jax.lax.stop_gradient(pix)).astype(jnp.int32)
    idx = jnp.where(x[..., 2:] > 0, idx, -1)

    mesh = plsc.VectorSubcoreMesh(core_axis_name="c", subcore_axis_name="s")
    out1 = pl.kernel(
        _sc_body,
        out_type=jax.ShapeDtypeStruct((BATCH * CH * NB,), jnp.float32),
        mesh=mesh,
        compiler_params=pltpu.CompilerParams(
            needs_layout_passes=False, use_tc_tiling_on_sc=False),
        scratch_types=[
            pltpu.VMEM_SHARED((NB,), jnp.float32),
            pltpu.VMEM_SHARED((NB,), jnp.float32),
            pltpu.VMEM_SHARED((NB,), jnp.float32),
            pltpu.VMEM_SHARED((NB,), jnp.float32),
            pltpu.VMEM_SHARED((NB,), jnp.float32),
            pltpu.VMEM((CHUNK * 8,), jnp.float32),
            pltpu.VMEM((CHUNK * 2,), jnp.int32),
            pltpu.VMEM((CHUNK,), jnp.int32),
            pltpu.VMEM((CHUNK,), jnp.float32),
            pltpu.VMEM((CHUNK,), jnp.float32),
            pltpu.VMEM((CHUNK,), jnp.float32),
            pltpu.VMEM((CHUNK,), jnp.float32),
            pltpu.VMEM((CHUNK,), jnp.float32),
            pltpu.VMEM((NSUB,), jnp.float32),
            pltpu.VMEM((NSUB,), jnp.float32),
            pltpu.VMEM((NSUB,), jnp.float32),
            pltpu.SemaphoreType.DMA,
        ],
    )(c.reshape(-1), idx.reshape(-1))

    out = pl.pallas_call(
        _retile_body,
        grid=(BATCH, CH, H // ROWS_PER_BLK),
        in_specs=[pl.BlockSpec(
            (ROWS_PER_BLK * W,),
            lambda b, ch, s: ((b * CH + ch) * (H // ROWS_PER_BLK) + s,))],
        out_specs=pl.BlockSpec(
            (1, 1, ROWS_PER_BLK, W), lambda b, ch, s: (b, ch, s, 0)),
        out_shape=jax.ShapeDtypeStruct((BATCH, CH, H, W), jnp.float32),
    )(out1)
    return out


# speedup vs baseline: 6.3373x; 2.1313x over previous
import functools
import jax
import jax.numpy as jnp
from jax import lax
from jax.experimental import pallas as pl
from jax.experimental.pallas import tpu as pltpu
from jax.experimental.pallas import tpu_sc as plsc

H = 512
W = 512
NB = H * W
NPTS = 131072
BATCH = 4
CH = 8
NC = 2
NS = 16
NPL = 10
CHUNK = 1024
PTS_PER_TILE = NPTS // NS
CHUNKS_PER_TILE = PTS_PER_TILE // CHUNK
GROUPS = CHUNK // 16
BINS_PER_TILE = NB // NS
ZC = 4096


def _sc_body(c_hbm, ind_hbm, cnt_hbm, out_hbm,
             acc0, acc1, acc2, acc3, acc4,
             c_a, c_b, i_a, i_b, n_a, n_b,
             v0a, v1a, v2a, v3a, v0b, v1b, v2b, v3b,
             zc_v, sin_a, sin_b, ssc_a, ssc_b, sout):
    core = lax.axis_index("c")
    sid = lax.axis_index("s")
    accs = (acc0, acc1, acc2, acc3, acc4)
    csets = (c_a, c_b)
    isets = (i_a, i_b)
    nsets = (n_a, n_b)
    vsets = ((v0a, v1a, v2a, v3a), (v0b, v1b, v2b, v3b))
    sins = (sin_a, sin_b)
    sscs = (ssc_a, ssc_b)

    nb_base = sid * BINS_PER_TILE

    def fire_in(b, chunk, q):
        base = sid * PTS_PER_TILE + chunk * CHUNK
        pbase = b * NPTS + base
        pltpu.async_copy(c_hbm.at[pl.ds(pbase * 8, CHUNK * 8)],
                         csets[q], sins[q])
        pltpu.async_copy(ind_hbm.at[pl.ds(pbase, CHUNK)], isets[q], sins[q])
        pltpu.async_copy(cnt_hbm.at[pl.ds(pbase, CHUNK)], nsets[q], sins[q])

    def wait_in(q):
        pltpu.make_async_copy(c_hbm.at[pl.ds(0, CHUNK * 8)],
                              csets[q], sins[q]).wait()
        pltpu.make_async_copy(ind_hbm.at[pl.ds(0, CHUNK)],
                              isets[q], sins[q]).wait()
        pltpu.make_async_copy(cnt_hbm.at[pl.ds(0, CHUNK)],
                              nsets[q], sins[q]).wait()

    def fire_sc(q):
        for p in range(4):
            pltpu.async_copy(vsets[q][p], accs[p].at[isets[q]],
                             sscs[q], add=True)
        pltpu.async_copy(nsets[q], accs[4].at[isets[q]], sscs[q], add=True)

    def drain_sc(q):
        for p in range(4):
            pltpu.make_async_copy(vsets[q][p], accs[p].at[isets[q]],
                                  sscs[q]).wait()
        pltpu.make_async_copy(nsets[q], accs[4].at[isets[q]], sscs[q]).wait()

    def _zc(i, _):
        zc_v[pl.ds(i * 16, 16)] = jnp.zeros((16,), jnp.float32)
        return 0
    lax.fori_loop(0, ZC // 16, _zc, 0)
    for p in range(5):
        for sub in range(BINS_PER_TILE // ZC):
            pltpu.async_copy(
                zc_v, accs[p].at[pl.ds(nb_base + sub * ZC, ZC)], sout)
    for p in range(5):
        for sub in range(BINS_PER_TILE // ZC):
            pltpu.make_async_copy(
                zc_v, accs[p].at[pl.ds(nb_base + sub * ZC, ZC)], sout).wait()
    plsc.subcore_barrier()

    def _batch(b, _):
        fire_in(b, 0, 0)

        def _outer(o, _):
            for par in range(2):
                q = par
                chunk = o * 2 + par
                wait_in(q)
                if par == 0:
                    @pl.when(o >= 1)
                    def _():
                        drain_sc(1)
                    fire_in(b, chunk + 1, 1)
                else:
                    @pl.when(o < (CHUNKS_PER_TILE // 2) - 1)
                    def _():
                        drain_sc(0)
                        fire_in(b, chunk + 1, 0)

                def _grp(g, _):
                    sl = pl.ds(g * 16, 16)
                    prow = g * 16 + lanes
                    m16 = nsets[q][sl]
                    for ch in range(4):
                        cval = plsc.load_gather(
                            csets[q], [prow * 8 + (core * 4 + ch)])
                        vsets[q][ch][sl] = cval * m16
                    return 0
                lanes = lax.iota(jnp.int32, 16)
                lax.fori_loop(0, GROUPS, _grp, 0)
                fire_sc(q)
            return 0
        lax.fori_loop(0, CHUNKS_PER_TILE // 2, _outer, 0)
        drain_sc(0)
        drain_sc(1)
        plsc.subcore_barrier()

        for p in range(5):
            pltpu.async_copy(
                accs[p].at[pl.ds(nb_base, BINS_PER_TILE)],
                out_hbm.at[pl.ds((b * NPL + core * 5 + p) * NB + nb_base,
                                 BINS_PER_TILE)],
                sout)
        for p in range(5):
            pltpu.make_async_copy(
                accs[p].at[pl.ds(nb_base, BINS_PER_TILE)],
                out_hbm.at[pl.ds((b * NPL + core * 5 + p) * NB + nb_base,
                                 BINS_PER_TILE)],
                sout).wait()
        for p in range(5):
            for sub in range(BINS_PER_TILE // ZC):
                pltpu.async_copy(
                    zc_v, accs[p].at[pl.ds(nb_base + sub * ZC, ZC)], sout)
        for p in range(5):
            for sub in range(BINS_PER_TILE // ZC):
                pltpu.make_async_copy(
                    zc_v, accs[p].at[pl.ds(nb_base + sub * ZC, ZC)],
                    sout).wait()
        plsc.subcore_barrier()
        return 0
    lax.fori_loop(0, BATCH, _batch, 0)


ROWS_PER_BLK = 128
BLK = ROWS_PER_BLK * W


def _norm_body(img_ref, cnt_ref, out_ref, r_ref):
    @pl.when(pl.program_id(2) == 0)
    def _():
        for i in range(ROWS_PER_BLK):
            r_ref[i, :] = 1.0 / jnp.maximum(cnt_ref[pl.ds(i * W, W)], 1.0)
    for i in range(ROWS_PER_BLK):
        out_ref[0, 0, i, :] = img_ref[pl.ds(i * W, W)] * r_ref[i, :]


@jax.jit
def kernel(x, c, K):
    cam = x / x[..., -1:]
    pix = jnp.matmul(cam, K.T)[..., :2]
    idx = jnp.round(jax.lax.stop_gradient(pix)).astype(jnp.int32)
    iu = idx[..., 0]
    iv = idx[..., 1]
    cond = ((iu > 0) & (iu < W) & (iv > 0) & (iv < H) & (x[..., 2] > 0))
    pid = jax.lax.broadcasted_iota(jnp.int32, iu.shape, 1)
    ind = jnp.where(cond, iu + iv * H, pid)
    cnt01 = cond.astype(jnp.float32)

    mesh = plsc.VectorSubcoreMesh(core_axis_name="c", subcore_axis_name="s")
    planes = pl.kernel(
        _sc_body,
        out_type=jax.ShapeDtypeStruct((BATCH * NPL * NB,), jnp.float32),
        mesh=mesh,
        compiler_params=pltpu.CompilerParams(
            needs_layout_passes=False, use_tc_tiling_on_sc=False),
        scratch_types=[
            pltpu.VMEM_SHARED((NB,), jnp.float32),
            pltpu.VMEM_SHARED((NB,), jnp.float32),
            pltpu.VMEM_SHARED((NB,), jnp.float32),
            pltpu.VMEM_SHARED((NB,), jnp.float32),
            pltpu.VMEM_SHARED((NB,), jnp.float32),
            pltpu.VMEM((CHUNK * 8,), jnp.float32),
            pltpu.VMEM((CHUNK * 8,), jnp.float32),
            pltpu.VMEM((CHUNK,), jnp.int32),
            pltpu.VMEM((CHUNK,), jnp.int32),
            pltpu.VMEM((CHUNK,), jnp.float32),
            pltpu.VMEM((CHUNK,), jnp.float32),
            pltpu.VMEM((CHUNK,), jnp.float32),
            pltpu.VMEM((CHUNK,), jnp.float32),
            pltpu.VMEM((CHUNK,), jnp.float32),
            pltpu.VMEM((CHUNK,), jnp.float32),
            pltpu.VMEM((CHUNK,), jnp.float32),
            pltpu.VMEM((CHUNK,), jnp.float32),
            pltpu.VMEM((CHUNK,), jnp.float32),
            pltpu.VMEM((CHUNK,), jnp.float32),
            pltpu.VMEM((ZC,), jnp.float32),
            pltpu.SemaphoreType.DMA,
            pltpu.SemaphoreType.DMA,
            pltpu.SemaphoreType.DMA,
            pltpu.SemaphoreType.DMA,
            pltpu.SemaphoreType.DMA,
        ],
    )(c.reshape(-1), ind.reshape(-1), cnt01.reshape(-1))

    out = pl.pallas_call(
        _norm_body,
        grid=(BATCH, H // ROWS_PER_BLK, CH),
        in_specs=[
            pl.BlockSpec(
                (BLK,),
                lambda b, s, ch: ((b * NPL + ch + ch // 4) * (NB // BLK) + s,)),
            pl.BlockSpec(
                (BLK,),
                lambda b, s, ch: ((b * NPL + 4) * (NB // BLK) + s,)),
        ],
        out_specs=pl.BlockSpec(
            (1, 1, ROWS_PER_BLK, W), lambda b, s, ch: (b, ch, s, 0)),
        out_shape=jax.ShapeDtypeStruct((BATCH, CH, H, W), jnp.float32),
        scratch_shapes=[pltpu.VMEM((ROWS_PER_BLK, W), jnp.float32)],
    )(planes, planes)
    return out


# speedup vs baseline: 10.8118x; 1.7061x over previous
import functools
import jax
import jax.numpy as jnp
from jax import lax
from jax.experimental import pallas as pl
from jax.experimental.pallas import tpu as pltpu
from jax.experimental.pallas import tpu_sc as plsc

H = 512
W = 512
NB = H * W
NPTS = 131072
BATCH = 4
CH = 8
NC = 2
NS = 16
NPL = 10
CHUNK = 1024
PTS_PER_TILE = NPTS // NS
CHUNKS_PER_TILE = PTS_PER_TILE // CHUNK
GROUPS = CHUNK // 16
BINS_PER_TILE = NB // NS
ZC = 4096


def _sc_body(c_hbm, ind_hbm, cnt_hbm, out_hbm,
             acc0, acc1, acc2, acc3, acc4,
             c_a, c_b, i_a, i_b, n_a, n_b,
             v0a, v1a, v2a, v3a, v0b, v1b, v2b, v3b,
             zc_v, sin_a, sin_b, ssc_a, ssc_b, sout):
    core = lax.axis_index("c")
    sid = lax.axis_index("s")
    accs = (acc0, acc1, acc2, acc3, acc4)
    csets = (c_a, c_b)
    isets = (i_a, i_b)
    nsets = (n_a, n_b)
    vsets = ((v0a, v1a, v2a, v3a), (v0b, v1b, v2b, v3b))
    sins = (sin_a, sin_b)
    sscs = (ssc_a, ssc_b)

    nb_base = sid * BINS_PER_TILE

    def fire_in(b, chunk, q):
        base = sid * PTS_PER_TILE + chunk * CHUNK
        pbase = b * NPTS + base
        pltpu.async_copy(c_hbm.at[pl.ds(pbase * 8, CHUNK * 8)],
                         csets[q], sins[q])
        pltpu.async_copy(ind_hbm.at[pl.ds(pbase, CHUNK)], isets[q], sins[q])
        pltpu.async_copy(cnt_hbm.at[pl.ds(pbase, CHUNK)], nsets[q], sins[q])

    def wait_in(q):
        pltpu.make_async_copy(c_hbm.at[pl.ds(0, CHUNK * 8)],
                              csets[q], sins[q]).wait()
        pltpu.make_async_copy(ind_hbm.at[pl.ds(0, CHUNK)],
                              isets[q], sins[q]).wait()
        pltpu.make_async_copy(cnt_hbm.at[pl.ds(0, CHUNK)],
                              nsets[q], sins[q]).wait()

    def fire_sc(q):
        for p in range(4):
            pltpu.async_copy(vsets[q][p], accs[p].at[isets[q]],
                             sscs[q], add=True)
        pltpu.async_copy(nsets[q], accs[4].at[isets[q]], sscs[q], add=True)

    def drain_sc(q):
        for p in range(4):
            pltpu.make_async_copy(vsets[q][p], accs[p].at[isets[q]],
                                  sscs[q]).wait()
        pltpu.make_async_copy(nsets[q], accs[4].at[isets[q]], sscs[q]).wait()

    def _zc(i, _):
        zc_v[pl.ds(i * 16, 16)] = jnp.zeros((16,), jnp.float32)
        return 0
    lax.fori_loop(0, ZC // 16, _zc, 0)
    for p in range(5):
        for sub in range(BINS_PER_TILE // ZC):
            pltpu.async_copy(
                zc_v, accs[p].at[pl.ds(nb_base + sub * ZC, ZC)], sout)
    for p in range(5):
        for sub in range(BINS_PER_TILE // ZC):
            pltpu.make_async_copy(
                zc_v, accs[p].at[pl.ds(nb_base + sub * ZC, ZC)], sout).wait()
    plsc.subcore_barrier()

    def _batch(b, _):
        fire_in(b, 0, 0)

        def _outer(o, _):
            for par in range(2):
                q = par
                chunk = o * 2 + par
                wait_in(q)
                if par == 0:
                    @pl.when(o >= 1)
                    def _():
                        drain_sc(1)
                    fire_in(b, chunk + 1, 1)
                else:
                    @pl.when(o < (CHUNKS_PER_TILE // 2) - 1)
                    def _():
                        drain_sc(0)
                        fire_in(b, chunk + 1, 0)

                def _grp(g, _):
                    sl = pl.ds(g * 16, 16)
                    cbase = (g // 8) * 1024 + (g % 8) * 16
                    m16 = nsets[q][sl]
                    for ch in range(4):
                        cval = csets[q][
                            pl.ds(cbase + (core * 4 + ch) * 128, 16)]
                        vsets[q][ch][sl] = cval * m16
                    return 0
                lax.fori_loop(0, GROUPS, _grp, 0)
                fire_sc(q)
            return 0
        lax.fori_loop(0, CHUNKS_PER_TILE // 2, _outer, 0)
        drain_sc(0)
        drain_sc(1)
        plsc.subcore_barrier()

        for p in range(5):
            pltpu.async_copy(
                accs[p].at[pl.ds(nb_base, BINS_PER_TILE)],
                out_hbm.at[pl.ds((b * NPL + core * 5 + p) * NB + nb_base,
                                 BINS_PER_TILE)],
                sout)
        for p in range(5):
            pltpu.make_async_copy(
                accs[p].at[pl.ds(nb_base, BINS_PER_TILE)],
                out_hbm.at[pl.ds((b * NPL + core * 5 + p) * NB + nb_base,
                                 BINS_PER_TILE)],
                sout).wait()
        for p in range(5):
            for sub in range(BINS_PER_TILE // ZC):
                pltpu.async_copy(
                    zc_v, accs[p].at[pl.ds(nb_base + sub * ZC, ZC)], sout)
        for p in range(5):
            for sub in range(BINS_PER_TILE // ZC):
                pltpu.make_async_copy(
                    zc_v, accs[p].at[pl.ds(nb_base + sub * ZC, ZC)],
                    sout).wait()
        plsc.subcore_barrier()
        return 0
    lax.fori_loop(0, BATCH, _batch, 0)


ROWS_PER_BLK = 128
BLK = ROWS_PER_BLK * W


def _norm_body(img_ref, cnt_ref, out_ref, r_ref):
    @pl.when(pl.program_id(2) == 0)
    def _():
        for i in range(ROWS_PER_BLK):
            r_ref[i, :] = 1.0 / jnp.maximum(cnt_ref[pl.ds(i * W, W)], 1.0)
    for i in range(ROWS_PER_BLK):
        out_ref[0, 0, i, :] = img_ref[pl.ds(i * W, W)] * r_ref[i, :]


@jax.jit
def kernel(x, c, K):
    cam = x / x[..., -1:]
    pix = jnp.matmul(cam, K.T)[..., :2]
    idx = jnp.round(jax.lax.stop_gradient(pix)).astype(jnp.int32)
    iu = idx[..., 0]
    iv = idx[..., 1]
    cond = ((iu > 0) & (iu < W) & (iv > 0) & (iv < H) & (x[..., 2] > 0))
    pid = jax.lax.broadcasted_iota(jnp.int32, iu.shape, 1)
    ind = jnp.where(cond, iu + iv * H, pid)
    cnt01 = cond.astype(jnp.float32)
    c_native = jnp.swapaxes(
        c.reshape(BATCH, NPTS // 128, 128, CH), 2, 3).reshape(-1)

    mesh = plsc.VectorSubcoreMesh(core_axis_name="c", subcore_axis_name="s")
    planes = pl.kernel(
        _sc_body,
        out_type=jax.ShapeDtypeStruct((BATCH * NPL * NB,), jnp.float32),
        mesh=mesh,
        compiler_params=pltpu.CompilerParams(
            needs_layout_passes=False, use_tc_tiling_on_sc=False),
        scratch_types=[
            pltpu.VMEM_SHARED((NB,), jnp.float32),
            pltpu.VMEM_SHARED((NB,), jnp.float32),
            pltpu.VMEM_SHARED((NB,), jnp.float32),
            pltpu.VMEM_SHARED((NB,), jnp.float32),
            pltpu.VMEM_SHARED((NB,), jnp.float32),
            pltpu.VMEM((CHUNK * 8,), jnp.float32),
            pltpu.VMEM((CHUNK * 8,), jnp.float32),
            pltpu.VMEM((CHUNK,), jnp.int32),
            pltpu.VMEM((CHUNK,), jnp.int32),
            pltpu.VMEM((CHUNK,), jnp.float32),
            pltpu.VMEM((CHUNK,), jnp.float32),
            pltpu.VMEM((CHUNK,), jnp.float32),
            pltpu.VMEM((CHUNK,), jnp.float32),
            pltpu.VMEM((CHUNK,), jnp.float32),
            pltpu.VMEM((CHUNK,), jnp.float32),
            pltpu.VMEM((CHUNK,), jnp.float32),
            pltpu.VMEM((CHUNK,), jnp.float32),
            pltpu.VMEM((CHUNK,), jnp.float32),
            pltpu.VMEM((CHUNK,), jnp.float32),
            pltpu.VMEM((ZC,), jnp.float32),
            pltpu.SemaphoreType.DMA,
            pltpu.SemaphoreType.DMA,
            pltpu.SemaphoreType.DMA,
            pltpu.SemaphoreType.DMA,
            pltpu.SemaphoreType.DMA,
        ],
    )(c_native, ind.reshape(-1), cnt01.reshape(-1))

    out = pl.pallas_call(
        _norm_body,
        grid=(BATCH, H // ROWS_PER_BLK, CH),
        in_specs=[
            pl.BlockSpec(
                (BLK,),
                lambda b, s, ch: ((b * NPL + ch + ch // 4) * (NB // BLK) + s,)),
            pl.BlockSpec(
                (BLK,),
                lambda b, s, ch: ((b * NPL + 4) * (NB // BLK) + s,)),
        ],
        out_specs=pl.BlockSpec(
            (1, 1, ROWS_PER_BLK, W), lambda b, s, ch: (b, ch, s, 0)),
        out_shape=jax.ShapeDtypeStruct((BATCH, CH, H, W), jnp.float32),
        scratch_shapes=[pltpu.VMEM((ROWS_PER_BLK, W), jnp.float32)],
    )(planes, planes)
    return out


# speedup vs baseline: 12.9950x; 1.2019x over previous
import functools
import jax
import jax.numpy as jnp
from jax import lax
from jax.experimental import pallas as pl
from jax.experimental.pallas import tpu as pltpu
from jax.experimental.pallas import tpu_sc as plsc

H = 512
W = 512
NB = H * W
NPTS = 131072
BATCH = 4
CH = 8
NC = 2
NS = 16
NPL = 10
CHUNK = 1024
PTS_PER_TILE = NPTS // NS
CHUNKS_PER_TILE = PTS_PER_TILE // CHUNK
GROUPS = CHUNK // 16
BINS_PER_TILE = NB // NS
ZC = 4096


def _sc_body(c_hbm, ind_hbm, cnt_hbm, out_hbm,
             acc0, acc1, acc2, acc3, acc4,
             c_a, c_b, i_a, i_b, n_a, n_b,
             v0a, v1a, v2a, v3a, v0b, v1b, v2b, v3b,
             zc_v, sin_a, sin_b, ssc_a, ssc_b, sout):
    core = lax.axis_index("c")
    sid = lax.axis_index("s")
    accs = (acc0, acc1, acc2, acc3, acc4)
    csets = (c_a, c_b)
    isets = (i_a, i_b)
    nsets = (n_a, n_b)
    vsets = ((v0a, v1a, v2a, v3a), (v0b, v1b, v2b, v3b))
    sins = (sin_a, sin_b)
    sscs = (ssc_a, ssc_b)

    nb_base = sid * BINS_PER_TILE

    def fire_in(b, chunk, q):
        base = sid * PTS_PER_TILE + chunk * CHUNK
        pbase = b * NPTS + base
        pltpu.async_copy(c_hbm.at[pl.ds(pbase * 8, CHUNK * 8)],
                         csets[q], sins[q])
        pltpu.async_copy(ind_hbm.at[pl.ds(pbase, CHUNK)], isets[q], sins[q])
        pltpu.async_copy(cnt_hbm.at[pl.ds(pbase, CHUNK)], nsets[q], sins[q])

    def wait_in(q):
        pltpu.make_async_copy(c_hbm.at[pl.ds(0, CHUNK * 8)],
                              csets[q], sins[q]).wait()
        pltpu.make_async_copy(ind_hbm.at[pl.ds(0, CHUNK)],
                              isets[q], sins[q]).wait()
        pltpu.make_async_copy(cnt_hbm.at[pl.ds(0, CHUNK)],
                              nsets[q], sins[q]).wait()

    def fire_sc(q):
        for p in range(4):
            pltpu.async_copy(vsets[q][p], accs[p].at[isets[q]],
                             sscs[q], add=True)
        pltpu.async_copy(nsets[q], accs[4].at[isets[q]], sscs[q], add=True)

    def drain_sc(q):
        for p in range(4):
            pltpu.make_async_copy(vsets[q][p], accs[p].at[isets[q]],
                                  sscs[q]).wait()
        pltpu.make_async_copy(nsets[q], accs[4].at[isets[q]], sscs[q]).wait()

    def _zc(i, _):
        zc_v[pl.ds(i * 16, 16)] = jnp.zeros((16,), jnp.float32)
        return 0
    lax.fori_loop(0, ZC // 16, _zc, 0)
    for p in range(5):
        for sub in range(BINS_PER_TILE // ZC):
            pltpu.async_copy(
                zc_v, accs[p].at[pl.ds(nb_base + sub * ZC, ZC)], sout)
    for p in range(5):
        for sub in range(BINS_PER_TILE // ZC):
            pltpu.make_async_copy(
                zc_v, accs[p].at[pl.ds(nb_base + sub * ZC, ZC)], sout).wait()
    plsc.subcore_barrier()

    def _batch(b, _):
        fire_in(b, 0, 0)

        def _outer(o, _):
            for par in range(2):
                q = par
                chunk = o * 2 + par
                wait_in(q)
                if par == 0:
                    @pl.when(o >= 1)
                    def _():
                        drain_sc(1)
                    fire_in(b, chunk + 1, 1)
                else:
                    @pl.when(o < (CHUNKS_PER_TILE // 2) - 1)
                    def _():
                        drain_sc(0)
                        fire_in(b, chunk + 1, 0)

                def _grp(g, _):
                    sl = pl.ds(g * 16, 16)
                    cbase = (g // 8) * 1024 + (g % 8) * 16
                    m16 = nsets[q][sl]
                    for ch in range(4):
                        cval = csets[q][
                            pl.ds(cbase + (core * 4 + ch) * 128, 16)]
                        vsets[q][ch][sl] = cval * m16
                    return 0
                lax.fori_loop(0, GROUPS, _grp, 0)
                fire_sc(q)
            return 0
        lax.fori_loop(0, CHUNKS_PER_TILE // 2, _outer, 0)
        drain_sc(0)
        drain_sc(1)
        plsc.subcore_barrier()

        for p in range(5):
            pltpu.async_copy(
                accs[p].at[pl.ds(nb_base, BINS_PER_TILE)],
                out_hbm.at[pl.ds((b * NPL + core * 5 + p) * NB + nb_base,
                                 BINS_PER_TILE)],
                sout)
        for p in range(5):
            pltpu.make_async_copy(
                accs[p].at[pl.ds(nb_base, BINS_PER_TILE)],
                out_hbm.at[pl.ds((b * NPL + core * 5 + p) * NB + nb_base,
                                 BINS_PER_TILE)],
                sout).wait()
        for p in range(5):
            for sub in range(BINS_PER_TILE // ZC):
                pltpu.async_copy(
                    zc_v, accs[p].at[pl.ds(nb_base + sub * ZC, ZC)], sout)
        for p in range(5):
            for sub in range(BINS_PER_TILE // ZC):
                pltpu.make_async_copy(
                    zc_v, accs[p].at[pl.ds(nb_base + sub * ZC, ZC)],
                    sout).wait()
        plsc.subcore_barrier()
        return 0
    lax.fori_loop(0, BATCH, _batch, 0)


ROWS_PER_BLK = 512
BLK = ROWS_PER_BLK * W


def _norm_body(img_ref, cnt_ref, out_ref, r_ref):
    @pl.when(pl.program_id(2) == 0)
    def _():
        for i in range(ROWS_PER_BLK):
            r_ref[i, :] = 1.0 / jnp.maximum(cnt_ref[pl.ds(i * W, W)], 1.0)
    for i in range(ROWS_PER_BLK):
        out_ref[0, 0, i, :] = img_ref[pl.ds(i * W, W)] * r_ref[i, :]


@jax.jit
def kernel(x, c, K):
    cam = x / x[..., -1:]
    pix = jnp.matmul(cam, K.T)[..., :2]
    idx = jnp.round(jax.lax.stop_gradient(pix)).astype(jnp.int32)
    iu = idx[..., 0]
    iv = idx[..., 1]
    cond = ((iu > 0) & (iu < W) & (iv > 0) & (iv < H) & (x[..., 2] > 0))
    pid = jax.lax.broadcasted_iota(jnp.int32, iu.shape, 1)
    ind = jnp.where(cond, iu + iv * H, pid)
    cnt01 = cond.astype(jnp.float32)
    c_native = jnp.swapaxes(
        c.reshape(BATCH, NPTS // 128, 128, CH), 2, 3).reshape(-1)

    mesh = plsc.VectorSubcoreMesh(core_axis_name="c", subcore_axis_name="s")
    planes = pl.kernel(
        _sc_body,
        out_type=jax.ShapeDtypeStruct((BATCH * NPL * NB,), jnp.float32),
        mesh=mesh,
        compiler_params=pltpu.CompilerParams(
            needs_layout_passes=False, use_tc_tiling_on_sc=False),
        scratch_types=[
            pltpu.VMEM_SHARED((NB,), jnp.float32),
            pltpu.VMEM_SHARED((NB,), jnp.float32),
            pltpu.VMEM_SHARED((NB,), jnp.float32),
            pltpu.VMEM_SHARED((NB,), jnp.float32),
            pltpu.VMEM_SHARED((NB,), jnp.float32),
            pltpu.VMEM((CHUNK * 8,), jnp.float32),
            pltpu.VMEM((CHUNK * 8,), jnp.float32),
            pltpu.VMEM((CHUNK,), jnp.int32),
            pltpu.VMEM((CHUNK,), jnp.int32),
            pltpu.VMEM((CHUNK,), jnp.float32),
            pltpu.VMEM((CHUNK,), jnp.float32),
            pltpu.VMEM((CHUNK,), jnp.float32),
            pltpu.VMEM((CHUNK,), jnp.float32),
            pltpu.VMEM((CHUNK,), jnp.float32),
            pltpu.VMEM((CHUNK,), jnp.float32),
            pltpu.VMEM((CHUNK,), jnp.float32),
            pltpu.VMEM((CHUNK,), jnp.float32),
            pltpu.VMEM((CHUNK,), jnp.float32),
            pltpu.VMEM((CHUNK,), jnp.float32),
            pltpu.VMEM((ZC,), jnp.float32),
            pltpu.SemaphoreType.DMA,
            pltpu.SemaphoreType.DMA,
            pltpu.SemaphoreType.DMA,
            pltpu.SemaphoreType.DMA,
            pltpu.SemaphoreType.DMA,
        ],
    )(c_native, ind.reshape(-1), cnt01.reshape(-1))

    out = pl.pallas_call(
        _norm_body,
        grid=(BATCH, H // ROWS_PER_BLK, CH),
        in_specs=[
            pl.BlockSpec(
                (BLK,),
                lambda b, s, ch: ((b * NPL + ch + ch // 4) * (NB // BLK) + s,)),
            pl.BlockSpec(
                (BLK,),
                lambda b, s, ch: ((b * NPL + 4) * (NB // BLK) + s,)),
        ],
        out_specs=pl.BlockSpec(
            (1, 1, ROWS_PER_BLK, W), lambda b, s, ch: (b, ch, s, 0)),
        out_shape=jax.ShapeDtypeStruct((BATCH, CH, H, W), jnp.float32),
        scratch_shapes=[pltpu.VMEM((ROWS_PER_BLK, W), jnp.float32)],
    )(planes, planes)
    return out


# speedup vs baseline: 13.0579x; 1.0048x over previous
import jax
import jax.numpy as jnp
from jax import lax
from jax.experimental import pallas as pl
from jax.experimental.pallas import tpu as pltpu
from jax.experimental.pallas import tpu_sc as plsc

H = 512
W = 512
NB = H * W
NPTS = 131072
BATCH = 4
CH = 8
NC = 2
NS = 16
NPL = 10
CHUNK = 1024
PTS_PER_TILE = NPTS // NS
CHUNKS_PER_TILE = PTS_PER_TILE // CHUNK
GROUPS = CHUNK // 16
BINS_PER_TILE = NB // NS
ZC = 4096


def _sc_body(c_hbm, ind_hbm, cnt_hbm, out_hbm,
             acc0, acc1, acc2, acc3, acc4,
             c_a, c_b, i_a, i_b, n_a, n_b,
             v0a, v1a, v2a, v3a, v0b, v1b, v2b, v3b,
             zc_v, sin_a, sin_b, ssc_a, ssc_b, sout):
    core = lax.axis_index("c")
    sid = lax.axis_index("s")
    accs = (acc0, acc1, acc2, acc3, acc4)
    csets = (c_a, c_b)
    isets = (i_a, i_b)
    nsets = (n_a, n_b)
    vsets = ((v0a, v1a, v2a, v3a), (v0b, v1b, v2b, v3b))
    sins = (sin_a, sin_b)
    sscs = (ssc_a, ssc_b)

    nb_base = sid * BINS_PER_TILE

    def fire_in(b, chunk, q):
        base = sid * PTS_PER_TILE + chunk * CHUNK
        pbase = b * NPTS + base
        pltpu.async_copy(c_hbm.at[pl.ds(pbase * 8, CHUNK * 8)],
                         csets[q], sins[q])
        pltpu.async_copy(ind_hbm.at[pl.ds(pbase, CHUNK)], isets[q], sins[q])
        pltpu.async_copy(cnt_hbm.at[pl.ds(pbase, CHUNK)], nsets[q], sins[q])

    def wait_in(q):
        pltpu.make_async_copy(c_hbm.at[pl.ds(0, CHUNK * 8)],
                              csets[q], sins[q]).wait()
        pltpu.make_async_copy(ind_hbm.at[pl.ds(0, CHUNK)],
                              isets[q], sins[q]).wait()
        pltpu.make_async_copy(cnt_hbm.at[pl.ds(0, CHUNK)],
                              nsets[q], sins[q]).wait()

    def fire_sc(q):
        for p in range(4):
            pltpu.async_copy(vsets[q][p], accs[p].at[isets[q]],
                             sscs[q], add=True)
        pltpu.async_copy(nsets[q], accs[4].at[isets[q]], sscs[q], add=True)

    def drain_sc(q):
        for p in range(4):
            pltpu.make_async_copy(vsets[q][p], accs[p].at[isets[q]],
                                  sscs[q]).wait()
        pltpu.make_async_copy(nsets[q], accs[4].at[isets[q]], sscs[q]).wait()

    def _zc(i, _):
        zc_v[pl.ds(i * 16, 16)] = jnp.zeros((16,), jnp.float32)
        return 0
    lax.fori_loop(0, ZC // 16, _zc, 0)
    for p in range(5):
        for sub in range(BINS_PER_TILE // ZC):
            pltpu.async_copy(
                zc_v, accs[p].at[pl.ds(nb_base + sub * ZC, ZC)], sout)
    for p in range(5):
        for sub in range(BINS_PER_TILE // ZC):
            pltpu.make_async_copy(
                zc_v, accs[p].at[pl.ds(nb_base + sub * ZC, ZC)], sout).wait()
    plsc.subcore_barrier()

    def _batch(b, _):
        fire_in(b, 0, 0)

        def _outer(o, _):
            for par in range(2):
                q = par
                chunk = o * 2 + par
                wait_in(q)
                if par == 0:
                    @pl.when(o >= 1)
                    def _():
                        drain_sc(1)
                    fire_in(b, chunk + 1, 1)
                else:
                    @pl.when(o < (CHUNKS_PER_TILE // 2) - 1)
                    def _():
                        drain_sc(0)
                        fire_in(b, chunk + 1, 0)

                def _grp(g, _):
                    sl = pl.ds(g * 16, 16)
                    cbase = (g // 8) * 1024 + (g % 8) * 16
                    m16 = nsets[q][sl]
                    for ch in range(4):
                        cval = csets[q][
                            pl.ds(cbase + (core * 4 + ch) * 128, 16)]
                        vsets[q][ch][sl] = cval * m16
                    return 0
                lax.fori_loop(0, GROUPS, _grp, 0)
                fire_sc(q)
            return 0
        lax.fori_loop(0, CHUNKS_PER_TILE // 2, _outer, 0)
        drain_sc(0)
        drain_sc(1)
        plsc.subcore_barrier()

        for p in range(5):
            pltpu.async_copy(
                accs[p].at[pl.ds(nb_base, BINS_PER_TILE)],
                out_hbm.at[pl.ds((b * NPL + core * 5 + p) * NB + nb_base,
                                 BINS_PER_TILE)],
                sout)
        for p in range(5):
            pltpu.make_async_copy(
                accs[p].at[pl.ds(nb_base, BINS_PER_TILE)],
                out_hbm.at[pl.ds((b * NPL + core * 5 + p) * NB + nb_base,
                                 BINS_PER_TILE)],
                sout).wait()
        for p in range(5):
            for sub in range(BINS_PER_TILE // ZC):
                pltpu.async_copy(
                    zc_v, accs[p].at[pl.ds(nb_base + sub * ZC, ZC)], sout)
        for p in range(5):
            for sub in range(BINS_PER_TILE // ZC):
                pltpu.make_async_copy(
                    zc_v, accs[p].at[pl.ds(nb_base + sub * ZC, ZC)],
                    sout).wait()
        plsc.subcore_barrier()
        return 0
    lax.fori_loop(0, BATCH, _batch, 0)


ROWS_PER_BLK = 512
BLK = ROWS_PER_BLK * W


def _norm_body(img_ref, cnt_ref, out_ref, r_ref):
    @pl.when(pl.program_id(2) == 0)
    def _():
        for i in range(ROWS_PER_BLK):
            r_ref[i, :] = 1.0 / jnp.maximum(cnt_ref[pl.ds(i * W, W)], 1.0)
    for i in range(ROWS_PER_BLK):
        out_ref[0, 0, i, :] = img_ref[pl.ds(i * W, W)] * r_ref[i, :]


@jax.jit
def kernel(x, c, K):
    cam = x / x[..., -1:]
    pix = jnp.matmul(cam, K.T)[..., :2]
    idx = jnp.round(jax.lax.stop_gradient(pix)).astype(jnp.int32)
    iu = idx[..., 0]
    iv = idx[..., 1]
    cond = ((iu > 0) & (iu < W) & (iv > 0) & (iv < H) & (x[..., 2] > 0))
    pid = jax.lax.broadcasted_iota(jnp.int32, iu.shape, 1)
    ind = jnp.where(cond, iu + iv * H, pid)
    cnt01 = cond.astype(jnp.float32)
    c_native = jnp.swapaxes(
        c.reshape(BATCH, NPTS // 128, 128, CH), 2, 3).reshape(-1)

    mesh = plsc.VectorSubcoreMesh(core_axis_name="c", subcore_axis_name="s")
    planes = pl.kernel(
        _sc_body,
        out_type=jax.ShapeDtypeStruct((BATCH * NPL * NB,), jnp.float32),
        mesh=mesh,
        compiler_params=pltpu.CompilerParams(
            needs_layout_passes=False, use_tc_tiling_on_sc=False),
        scratch_types=[
            pltpu.VMEM_SHARED((NB,), jnp.float32),
            pltpu.VMEM_SHARED((NB,), jnp.float32),
            pltpu.VMEM_SHARED((NB,), jnp.float32),
            pltpu.VMEM_SHARED((NB,), jnp.float32),
            pltpu.VMEM_SHARED((NB,), jnp.float32),
            pltpu.VMEM((CHUNK * 8,), jnp.float32),
            pltpu.VMEM((CHUNK * 8,), jnp.float32),
            pltpu.VMEM((CHUNK,), jnp.int32),
            pltpu.VMEM((CHUNK,), jnp.int32),
            pltpu.VMEM((CHUNK,), jnp.float32),
            pltpu.VMEM((CHUNK,), jnp.float32),
            pltpu.VMEM((CHUNK,), jnp.float32),
            pltpu.VMEM((CHUNK,), jnp.float32),
            pltpu.VMEM((CHUNK,), jnp.float32),
            pltpu.VMEM((CHUNK,), jnp.float32),
            pltpu.VMEM((CHUNK,), jnp.float32),
            pltpu.VMEM((CHUNK,), jnp.float32),
            pltpu.VMEM((CHUNK,), jnp.float32),
            pltpu.VMEM((CHUNK,), jnp.float32),
            pltpu.VMEM((ZC,), jnp.float32),
            pltpu.SemaphoreType.DMA,
            pltpu.SemaphoreType.DMA,
            pltpu.SemaphoreType.DMA,
            pltpu.SemaphoreType.DMA,
            pltpu.SemaphoreType.DMA,
        ],
    )(c_native, ind.reshape(-1), cnt01.reshape(-1))

    out = pl.pallas_call(
        _norm_body,
        grid=(BATCH, H // ROWS_PER_BLK, CH),
        in_specs=[
            pl.BlockSpec(
                (BLK,),
                lambda b, s, ch: ((b * NPL + ch + ch // 4) * (NB // BLK) + s,)),
            pl.BlockSpec(
                (BLK,),
                lambda b, s, ch: ((b * NPL + 4) * (NB // BLK) + s,)),
        ],
        out_specs=pl.BlockSpec(
            (1, 1, ROWS_PER_BLK, W), lambda b, s, ch: (b, ch, s, 0)),
        out_shape=jax.ShapeDtypeStruct((BATCH, CH, H, W), jnp.float32),
        scratch_shapes=[pltpu.VMEM((ROWS_PER_BLK, W), jnp.float32)],
    )(planes, planes)
    return out
